# Initial kernel scaffold; baseline (speedup 1.0000x reference)
import jax
import jax.numpy as jnp
from jax.experimental import pallas as pl


def kernel(x, adj, weight, bias, W1, b1, W2, b2, W3, b3, W4, b4):
    raise NotImplementedError("write your pallas kernel here")


# speedup vs baseline: 1.1578x; 1.1578x over previous
import jax
import jax.numpy as jnp
from jax.experimental import pallas as pl
from jax.experimental.pallas import tpu as pltpu

N = 10000
F = 128
H = 128
C = 64
BR1 = 200
BR = 1000


def _pass1_body(adj_ref, xc_ref, xr_ref, w_ref, b_ref, w1_ref, b1_ref,
                adj16_ref, x1_ref):
    a16 = adj_ref[...].astype(jnp.bfloat16)
    adj16_ref[...] = a16
    acc = jnp.dot(a16, xc_ref[...], preferred_element_type=jnp.float32)
    z = jnp.dot(xr_ref[...].astype(jnp.bfloat16), w_ref[...],
                preferred_element_type=jnp.float32) + b_ref[...]
    h = jnp.dot(acc.astype(jnp.bfloat16), w1_ref[...],
                preferred_element_type=jnp.float32) + b1_ref[...]
    x1_ref[...] = jnp.maximum(h, 0.0) + z


def _mid_body(adj16_ref, hc_ref, hr_ref, w_ref, b_ref, xn_ref, a_ref):
    acc = jnp.dot(adj16_ref[...], hc_ref[...],
                  preferred_element_type=jnp.float32)
    a_ref[...] = acc
    g = jnp.dot(acc.astype(jnp.bfloat16), w_ref[...],
                preferred_element_type=jnp.float32) + b_ref[...]
    xn_ref[...] = jnp.maximum(g, 0.0) + hr_ref[...]


def _last_body(adj16_ref, hc_ref, a1_ref, a2_ref, w4a_ref, w4b_ref,
               w4c_ref, b4_ref, out_ref):
    acc = jnp.dot(adj16_ref[...], hc_ref[...],
                  preferred_element_type=jnp.float32)
    x4 = (jnp.dot(acc.astype(jnp.bfloat16), w4a_ref[...],
                  preferred_element_type=jnp.float32)
          + jnp.dot(a2_ref[...].astype(jnp.bfloat16), w4b_ref[...],
                    preferred_element_type=jnp.float32)
          + jnp.dot(a1_ref[...].astype(jnp.bfloat16), w4c_ref[...],
                    preferred_element_type=jnp.float32)
          + b4_ref[...])
    m = jnp.max(x4, axis=1, keepdims=True)
    lse = m + jnp.log(jnp.sum(jnp.exp(x4 - m), axis=1, keepdims=True))
    out_ref[...] = x4 - lse


def _row_spec(br, width):
    return pl.BlockSpec((br, width), lambda i: (i, 0))


def _whole_spec(rows, cols):
    return pl.BlockSpec((rows, cols), lambda i: (0, 0))


_PARAMS = pltpu.CompilerParams(dimension_semantics=("arbitrary",))


def kernel(x, adj, weight, bias, W1, b1, W2, b2, W3, b3, W4, b4):
    xc = x.astype(jnp.bfloat16)

    adj16, x1 = pl.pallas_call(
        _pass1_body,
        grid=(N // BR1,),
        in_specs=[
            _row_spec(BR1, N),
            _whole_spec(N, F),
            _row_spec(BR1, F),
            _whole_spec(F, H),
            _whole_spec(1, H),
            _whole_spec(F, H),
            _whole_spec(1, H),
        ],
        out_specs=[_row_spec(BR1, N), _row_spec(BR1, H)],
        out_shape=[
            jax.ShapeDtypeStruct((N, N), jnp.bfloat16),
            jax.ShapeDtypeStruct((N, H), jnp.float32),
        ],
        compiler_params=_PARAMS,
    )(adj, xc, x, weight.astype(jnp.bfloat16), bias.reshape(1, H),
      W1.astype(jnp.bfloat16), b1.reshape(1, H))

    def mid(h, W, b):
        return pl.pallas_call(
            _mid_body,
            grid=(N // BR,),
            in_specs=[
                _row_spec(BR, N),
                _whole_spec(N, H),
                _row_spec(BR, H),
                _whole_spec(H, H),
                _whole_spec(1, H),
            ],
            out_specs=[_row_spec(BR, H), _row_spec(BR, H)],
            out_shape=[
                jax.ShapeDtypeStruct((N, H), jnp.float32),
                jax.ShapeDtypeStruct((N, H), jnp.float32),
            ],
            compiler_params=_PARAMS,
        )(adj16, h.astype(jnp.bfloat16), h, W.astype(jnp.bfloat16),
          b.reshape(1, H))

    x2, A1 = mid(x1, W2, b2)
    x3, A2 = mid(x2, W3, b3)

    out = pl.pallas_call(
        _last_body,
        grid=(N // BR,),
        in_specs=[
            _row_spec(BR, N),
            _whole_spec(N, H),
            _row_spec(BR, H),
            _row_spec(BR, H),
            _whole_spec(H, C),
            _whole_spec(H, C),
            _whole_spec(H, C),
            _whole_spec(1, C),
        ],
        out_specs=_row_spec(BR, C),
        out_shape=jax.ShapeDtypeStruct((N, C), jnp.float32),
        compiler_params=_PARAMS,
    )(adj16, x3.astype(jnp.bfloat16), A1, A2,
      W4[:H].astype(jnp.bfloat16), W4[H:2 * H].astype(jnp.bfloat16),
      W4[2 * H:].astype(jnp.bfloat16), b4.reshape(1, C))

    return out


# speedup vs baseline: 1.1583x; 1.0004x over previous
import jax
import jax.numpy as jnp
from jax.experimental import pallas as pl
from jax.experimental.pallas import tpu as pltpu

N = 10000
F = 128
H = 128
C = 64
BR1 = 200
BR = 1000


def _pass1_body(adj_ref, xc_ref, xr_ref, w_ref, b_ref, w1_ref, b1_ref,
                adj16_ref, x1_ref):
    a16 = adj_ref[...].astype(jnp.bfloat16)
    adj16_ref[...] = a16
    acc = jnp.dot(a16, xc_ref[...], preferred_element_type=jnp.float32)
    z = jnp.dot(xr_ref[...].astype(jnp.bfloat16), w_ref[...],
                preferred_element_type=jnp.float32) + b_ref[...]
    h = jnp.dot(acc.astype(jnp.bfloat16), w1_ref[...],
                preferred_element_type=jnp.float32) + b1_ref[...]
    x1_ref[...] = jnp.maximum(h, 0.0) + z


def _mid_body(adj16_ref, hc_ref, hr_ref, w_ref, b_ref, xn_ref, a_ref):
    acc = jnp.dot(adj16_ref[...], hc_ref[...],
                  preferred_element_type=jnp.float32)
    a_ref[...] = acc
    g = jnp.dot(acc.astype(jnp.bfloat16), w_ref[...],
                preferred_element_type=jnp.float32) + b_ref[...]
    xn_ref[...] = jnp.maximum(g, 0.0) + hr_ref[...]


def _last_body(adj16_ref, hc_ref, a1_ref, a2_ref, w4a_ref, w4b_ref,
               w4c_ref, b4_ref, out_ref):
    acc = jnp.dot(adj16_ref[...], hc_ref[...],
                  preferred_element_type=jnp.float32)
    x4 = (jnp.dot(acc.astype(jnp.bfloat16), w4a_ref[...],
                  preferred_element_type=jnp.float32)
          + jnp.dot(a2_ref[...].astype(jnp.bfloat16), w4b_ref[...],
                    preferred_element_type=jnp.float32)
          + jnp.dot(a1_ref[...].astype(jnp.bfloat16), w4c_ref[...],
                    preferred_element_type=jnp.float32)
          + b4_ref[...])
    m = jnp.max(x4, axis=1, keepdims=True)
    lse = m + jnp.log(jnp.sum(jnp.exp(x4 - m), axis=1, keepdims=True))
    out_ref[...] = x4 - lse


def _row_spec(br, width):
    return pl.BlockSpec((br, width), lambda i: (i, 0))


def _whole_spec(rows, cols):
    return pl.BlockSpec((rows, cols), lambda i: (0, 0))


_PARAMS = pltpu.CompilerParams(dimension_semantics=("parallel",))


def kernel(x, adj, weight, bias, W1, b1, W2, b2, W3, b3, W4, b4):
    xc = x.astype(jnp.bfloat16)

    adj16, x1 = pl.pallas_call(
        _pass1_body,
        grid=(N // BR1,),
        in_specs=[
            _row_spec(BR1, N),
            _whole_spec(N, F),
            _row_spec(BR1, F),
            _whole_spec(F, H),
            _whole_spec(1, H),
            _whole_spec(F, H),
            _whole_spec(1, H),
        ],
        out_specs=[_row_spec(BR1, N), _row_spec(BR1, H)],
        out_shape=[
            jax.ShapeDtypeStruct((N, N), jnp.bfloat16),
            jax.ShapeDtypeStruct((N, H), jnp.float32),
        ],
        compiler_params=_PARAMS,
    )(adj, xc, x, weight.astype(jnp.bfloat16), bias.reshape(1, H),
      W1.astype(jnp.bfloat16), b1.reshape(1, H))

    def mid(h, W, b):
        return pl.pallas_call(
            _mid_body,
            grid=(N // BR,),
            in_specs=[
                _row_spec(BR, N),
                _whole_spec(N, H),
                _row_spec(BR, H),
                _whole_spec(H, H),
                _whole_spec(1, H),
            ],
            out_specs=[_row_spec(BR, H), _row_spec(BR, H)],
            out_shape=[
                jax.ShapeDtypeStruct((N, H), jnp.float32),
                jax.ShapeDtypeStruct((N, H), jnp.float32),
            ],
            compiler_params=_PARAMS,
        )(adj16, h.astype(jnp.bfloat16), h, W.astype(jnp.bfloat16),
          b.reshape(1, H))

    x2, A1 = mid(x1, W2, b2)
    x3, A2 = mid(x2, W3, b3)

    out = pl.pallas_call(
        _last_body,
        grid=(N // BR,),
        in_specs=[
            _row_spec(BR, N),
            _whole_spec(N, H),
            _row_spec(BR, H),
            _row_spec(BR, H),
            _whole_spec(H, C),
            _whole_spec(H, C),
            _whole_spec(H, C),
            _whole_spec(1, C),
        ],
        out_specs=_row_spec(BR, C),
        out_shape=jax.ShapeDtypeStruct((N, C), jnp.float32),
        compiler_params=_PARAMS,
    )(adj16, x3.astype(jnp.bfloat16), A1, A2,
      W4[:H].astype(jnp.bfloat16), W4[H:2 * H].astype(jnp.bfloat16),
      W4[2 * H:].astype(jnp.bfloat16), b4.reshape(1, C))

    return out


# speedup vs baseline: 1.1907x; 1.0280x over previous
import jax
import jax.numpy as jnp
from jax.experimental import pallas as pl
from jax.experimental.pallas import tpu as pltpu

N = 10000
F = 128
H = 128
C = 64
BR1 = 400
BR = 1000

_F32 = jnp.float32
_BF16 = jnp.bfloat16


def _pass1_body(adj_ref, xc_ref, xr_ref, w_ref, b_ref, w1_ref, b1_ref,
                adj16_ref, x1_ref):
    a16 = adj_ref[...].astype(_BF16)
    adj16_ref[...] = a16
    acc = jnp.dot(a16, xc_ref[...], preferred_element_type=_F32)
    z = jnp.dot(xr_ref[...], w_ref[...], preferred_element_type=_F32) \
        + b_ref[...]
    h = jnp.dot(acc.astype(_BF16), w1_ref[...], preferred_element_type=_F32) \
        + b1_ref[...]
    x1_ref[...] = (jnp.maximum(h, 0.0) + z).astype(_BF16)


def _mid_body(adj16_ref, hc_ref, hr_ref, w_ref, b_ref, xn_ref, a_ref):
    acc = jnp.dot(adj16_ref[...], hc_ref[...], preferred_element_type=_F32)
    a_ref[...] = acc.astype(_BF16)
    g = jnp.dot(acc.astype(_BF16), w_ref[...], preferred_element_type=_F32) \
        + b_ref[...]
    xn_ref[...] = (jnp.maximum(g, 0.0)
                   + hr_ref[...].astype(_F32)).astype(_BF16)


def _last_body(adj16_ref, hc_ref, a1_ref, a2_ref, w4a_ref, w4b_ref,
               w4c_ref, b4_ref, out_ref):
    acc = jnp.dot(adj16_ref[...], hc_ref[...], preferred_element_type=_F32)
    x4 = (jnp.dot(acc.astype(_BF16), w4a_ref[...],
                  preferred_element_type=_F32)
          + jnp.dot(a2_ref[...], w4b_ref[...], preferred_element_type=_F32)
          + jnp.dot(a1_ref[...], w4c_ref[...], preferred_element_type=_F32)
          + b4_ref[...])
    m = jnp.max(x4, axis=1, keepdims=True)
    lse = m + jnp.log(jnp.sum(jnp.exp(x4 - m), axis=1, keepdims=True))
    out_ref[...] = x4 - lse


def _row_spec(br, width):
    return pl.BlockSpec((br, width), lambda i: (i, 0))


def _whole_spec(rows, cols):
    return pl.BlockSpec((rows, cols), lambda i: (0, 0))


_PARAMS = pltpu.CompilerParams(dimension_semantics=("arbitrary",))


def kernel(x, adj, weight, bias, W1, b1, W2, b2, W3, b3, W4, b4):
    xc = x.astype(_BF16)

    adj16, x1 = pl.pallas_call(
        _pass1_body,
        grid=(N // BR1,),
        in_specs=[
            _row_spec(BR1, N),
            _whole_spec(N, F),
            _row_spec(BR1, F),
            _whole_spec(F, H),
            _whole_spec(1, H),
            _whole_spec(F, H),
            _whole_spec(1, H),
        ],
        out_specs=[_row_spec(BR1, N), _row_spec(BR1, H)],
        out_shape=[
            jax.ShapeDtypeStruct((N, N), _BF16),
            jax.ShapeDtypeStruct((N, H), _BF16),
        ],
        compiler_params=_PARAMS,
    )(adj, xc, xc, weight.astype(_BF16), bias.reshape(1, H),
      W1.astype(_BF16), b1.reshape(1, H))

    def mid(h, W, b):
        return pl.pallas_call(
            _mid_body,
            grid=(N // BR,),
            in_specs=[
                _row_spec(BR, N),
                _whole_spec(N, H),
                _row_spec(BR, H),
                _whole_spec(H, H),
                _whole_spec(1, H),
            ],
            out_specs=[_row_spec(BR, H), _row_spec(BR, H)],
            out_shape=[
                jax.ShapeDtypeStruct((N, H), _BF16),
                jax.ShapeDtypeStruct((N, H), _BF16),
            ],
            compiler_params=_PARAMS,
        )(adj16, h, h, W.astype(_BF16), b.reshape(1, H))

    x2, A1 = mid(x1, W2, b2)
    x3, A2 = mid(x2, W3, b3)

    out = pl.pallas_call(
        _last_body,
        grid=(N // BR,),
        in_specs=[
            _row_spec(BR, N),
            _whole_spec(N, H),
            _row_spec(BR, H),
            _row_spec(BR, H),
            _whole_spec(H, C),
            _whole_spec(H, C),
            _whole_spec(H, C),
            _whole_spec(1, C),
        ],
        out_specs=_row_spec(BR, C),
        out_shape=jax.ShapeDtypeStruct((N, C), _F32),
        compiler_params=_PARAMS,
    )(adj16, x3, A1, A2,
      W4[:H].astype(_BF16), W4[H:2 * H].astype(_BF16),
      W4[2 * H:].astype(_BF16), b4.reshape(1, C))

    return out
